# baseline (speedup 1.0000x reference)
import jax
import jax.numpy as jnp
from jax.experimental import pallas as pl

_DEPTH = 1000
_BLK = 64


def _one_hot_body(inp_ref, out_ref):
    inp = inp_ref[...]
    iota = jax.lax.broadcasted_iota(jnp.int32, (_BLK, 26, _DEPTH), 2)
    out_ref[...] = (iota == inp[:, :, None]).astype(jnp.float32)


def kernel(inputs):
    x = inputs.astype(jnp.int32)
    n = x.shape[0]
    return pl.pallas_call(
        _one_hot_body,
        grid=(n // _BLK,),
        in_specs=[pl.BlockSpec((_BLK, 26), lambda i: (i, 0))],
        out_specs=pl.BlockSpec((_BLK, 26, _DEPTH), lambda i: (i, 0, 0)),
        out_shape=jax.ShapeDtypeStruct((n, 26, _DEPTH), jnp.float32),
    )(x)
